# Initial kernel scaffold; baseline (speedup 1.0000x reference)
import jax
import jax.numpy as jnp
from jax.experimental import pallas as pl


def kernel(x, absolute_position_embed, year_embed, month_embed):
    raise NotImplementedError("write your pallas kernel here")


# speedup vs baseline: 8.4338x; 8.4338x over previous
import functools

import jax
import jax.numpy as jnp
from jax import lax
from jax.experimental import pallas as pl
from jax.experimental.pallas import tpu as pltpu
from jax.experimental.pallas import tpu_sc as plsc

B = 4096
L = 200
D = 64
N = B * L
NC = 2
NS = 16
NW = NC * NS
PER_W = N // NW
C = 512
N_CHUNKS = PER_W // C
LANES = 16


def _body(year_hbm, month_hbm, pos_hbm, iy_hbm, im_hbm, ip_hbm, out_hbm,
          iy_v, im_v, ip_v, acc_v, b1_v, b2_v, sem):
    cid = lax.axis_index("c")
    sid = lax.axis_index("s")
    wid = sid * NC + cid
    base = wid * PER_W

    def chunk_body(k, carry):
        off = pl.multiple_of(base + k * C, C)
        pltpu.sync_copy(iy_hbm.at[pl.ds(off, C)], iy_v)
        pltpu.sync_copy(im_hbm.at[pl.ds(off, C)], im_v)
        pltpu.sync_copy(ip_hbm.at[pl.ds(off, C)], ip_v)
        cp0 = pltpu.async_copy(year_hbm.at[iy_v], acc_v, sem)
        cp1 = pltpu.async_copy(month_hbm.at[im_v], b1_v, sem)
        cp2 = pltpu.async_copy(pos_hbm.at[ip_v], b2_v, sem)
        cp0.wait()
        cp1.wait()
        cp2.wait()

        def row_body(r, rcarry):
            for g in range(D // LANES):
                s = pl.ds(g * LANES, LANES)
                acc_v[r, s] = acc_v[r, s] + b1_v[r, s] + b2_v[r, s]
            return rcarry

        lax.fori_loop(0, C, row_body, 0, unroll=False)
        pltpu.sync_copy(acc_v, out_hbm.at[pl.ds(off, C)])
        return carry

    lax.fori_loop(0, N_CHUNKS, chunk_body, 0, unroll=False)


@jax.jit
def _temporal_embedding(year_embed, month_embed, pos_embed, iy, im, ip):
    run = pl.kernel(
        _body,
        out_type=jax.ShapeDtypeStruct((N, D), jnp.float32),
        mesh=plsc.VectorSubcoreMesh(core_axis_name="c", subcore_axis_name="s"),
        scratch_types=[
            pltpu.VMEM((C,), jnp.int32),
            pltpu.VMEM((C,), jnp.int32),
            pltpu.VMEM((C,), jnp.int32),
            pltpu.VMEM((C, D), jnp.float32),
            pltpu.VMEM((C, D), jnp.float32),
            pltpu.VMEM((C, D), jnp.float32),
            pltpu.SemaphoreType.DMA,
        ],
        compiler_params=pltpu.CompilerParams(use_tc_tiling_on_sc=False),
    )
    return run(year_embed, month_embed, pos_embed, iy, im, ip)


def kernel(x, absolute_position_embed, year_embed, month_embed):
    idx = x.astype(jnp.int32).reshape(N, 3)
    iy = idx[:, 0]
    im = idx[:, 1]
    ip = idx[:, 2]
    out = _temporal_embedding(year_embed, month_embed, absolute_position_embed,
                              iy, im, ip)
    return out.reshape(B, L, D)
